# baseline (speedup 1.0000x reference)
import functools

import jax
import jax.numpy as jnp
from jax import lax
from jax.experimental import pallas as pl
from jax.experimental.pallas import tpu as pltpu
from jax.experimental.pallas import tpu_sc as plsc

D = 64
SEQ = 200
L = 16
NC = 2
NS = 16
NW = NC * NS

BB = 128
JT = D // 8
ST = SEQ // 8
SCALE = 8.0


NGB = 4
NOB = 2


def _sc_body(idx_hbm, pos_hbm, tab_hbm, out_hbm,
             idx_v, pos_v, ihs, gbufs, obufs, gsems, osems):
    wid = lax.axis_index("s") * NC + lax.axis_index("c")
    pltpu.sync_copy(idx_hbm.at[pl.ds(0, ST), wid], idx_v)
    pltpu.sync_copy(pos_hbm, pos_v)

    iota = lax.iota(jnp.int32, L)
    rowids = [jnp.int32(k * L) + iota for k in range(BB // L)]

    def fire_gather(s, bank):
        st, sr = s // 8, s % 8
        for k in range(BB // L):
            sl = pl.ds(k * L, L)
            ihs[bank][sl] = idx_v[st, sr, sl] >> 1
        pltpu.async_copy(tab_hbm.at[ihs[bank]], gbufs[bank], gsems[bank])

    for b in range(NGB - 1):
        fire_gather(b, b)

    def do_block(st, sr):
        s = st * 8 + sr
        bank = sr % NGB
        gbuf, gsem = gbufs[bank], gsems[bank]
        ob = sr % NOB
        obuf, osem = obufs[ob], osems[ob]

        @pl.when(s + NGB - 1 < SEQ)
        def _():
            fire_gather(s + NGB - 1, (sr + NGB - 1) % NGB)

        pltpu.make_async_copy(tab_hbm.at[ihs[bank]], gbuf, gsem).wait()

        if sr >= NOB:
            pltpu.make_async_copy(obuf, out_hbm.at[s - NOB, pl.ds(0, JT), wid],
                                  osem).wait()
        else:
            @pl.when(st > 0)
            def _():
                pltpu.make_async_copy(obuf,
                                      out_hbm.at[s - NOB, pl.ds(0, JT), wid],
                                      osem).wait()

        halves = [(idx_v[st, sr, pl.ds(k * L, L)] & 1) * D
                  for k in range(BB // L)]
        pbase = s * D

        def dims(j, carry):
            a = pbase + j
            pj = plsc.load_gather(pos_v, [jnp.full((L,), a >> 7, jnp.int32),
                                          jnp.full((L,), a & 127, jnp.int32)])
            jt, jr = j >> 3, j & 7
            for k in range(BB // L):
                x = plsc.load_gather(gbuf, [rowids[k], halves[k] + j])
                obuf[jt, jr, pl.ds(k * L, L)] = x * SCALE + pj
            return carry
        lax.fori_loop(0, D, dims, 0, unroll=False)

        pltpu.async_copy(obuf, out_hbm.at[s, pl.ds(0, JT), wid], osem)

    def outer(st, carry):
        for sr in range(8):
            do_block(st, sr)
        return carry

    lax.fori_loop(0, ST, outer, 0, unroll=False)

    for t in range(NOB):
        s = SEQ - NOB + t
        pltpu.make_async_copy(obufs[s % NOB],
                              out_hbm.at[s, pl.ds(0, JT), wid],
                              osems[s % NOB]).wait()


def kernel(inputs, token_table, position_table):
    bsz, seq = inputs.shape
    vocab, d = token_table.shape
    nbb = bsz // BB

    idx_q = inputs.astype(jnp.int32).reshape(nbb, BB, seq // 8, 8).transpose(2, 0, 3, 1)
    tab2 = token_table.reshape(vocab // 2, 2 * d)
    pos2 = position_table.reshape(seq // 2, 2 * d)

    mesh = plsc.VectorSubcoreMesh(core_axis_name="c", subcore_axis_name="s")

    def body(idx_hbm, pos_hbm, tab_hbm, out_hbm, idx_v, pos_v,
             ih0, ih1, ih2, ih3, g0, g1, g2, g3, o0, o1,
             gs0, gs1, gs2, gs3, os0, os1):
        _sc_body(idx_hbm, pos_hbm, tab_hbm, out_hbm, idx_v, pos_v,
                 (ih0, ih1, ih2, ih3), (g0, g1, g2, g3), (o0, o1),
                 (gs0, gs1, gs2, gs3), (os0, os1))

    p5 = pl.kernel(
        body,
        out_type=jax.ShapeDtypeStruct((seq, JT, nbb, 8, BB), jnp.float32),
        mesh=mesh,
        compiler_params=pltpu.CompilerParams(use_tc_tiling_on_sc=True,
                                             needs_layout_passes=False),
        scratch_types=(
            [pltpu.VMEM((ST, 8, BB), jnp.int32),
             pltpu.VMEM((seq // 2, 2 * d), jnp.float32)]
            + [pltpu.VMEM((BB,), jnp.int32) for _ in range(NGB)]
            + [pltpu.VMEM((BB, 2 * d), jnp.float32) for _ in range(NGB)]
            + [pltpu.VMEM((JT, 8, BB), jnp.float32) for _ in range(NOB)]
            + [pltpu.SemaphoreType.DMA for _ in range(NGB + NOB)]
        ),
    )(idx_q, pos2, tab2)

    return p5.transpose(2, 4, 0, 1, 3).reshape(bsz, seq, d)


# speedup vs baseline: 1.6391x; 1.6391x over previous
import functools

import jax
import jax.numpy as jnp
from jax import lax
from jax.experimental import pallas as pl
from jax.experimental.pallas import tpu as pltpu
from jax.experimental.pallas import tpu_sc as plsc

D = 64
SEQ = 200
L = 16
NC = 2
NS = 16
NW = NC * NS

BB = 128
JT = D // 8
ST = SEQ // 8
SCALE = 8.0


NGB = 4
NOB = 2


def _sc_body(idx_hbm, pos_hbm, tab_hbm, out_hbm,
             idx_v, pos_v, ihs, gbufs, obufs, gsems, osems):
    wid = lax.axis_index("s") * NC + lax.axis_index("c")
    pltpu.sync_copy(idx_hbm.at[pl.ds(0, ST), wid], idx_v)
    pltpu.sync_copy(pos_hbm, pos_v)

    iota = lax.iota(jnp.int32, L)
    rowids = [jnp.int32(k * L) + iota for k in range(BB // L)]

    def fire_gather(s, bank):
        st, sr = s // 8, s % 8
        for k in range(BB // L):
            sl = pl.ds(k * L, L)
            ihs[bank][sl] = idx_v[st, sr, sl] >> 1
        pltpu.async_copy(tab_hbm.at[ihs[bank]], gbufs[bank], gsems[bank])

    for b in range(NGB - 1):
        fire_gather(b, b)

    def do_block(st, sr):
        s = st * 8 + sr
        bank = sr % NGB
        gbuf, gsem = gbufs[bank], gsems[bank]
        ob = sr % NOB
        obuf, osem = obufs[ob], osems[ob]

        @pl.when(s + NGB - 1 < SEQ)
        def _():
            fire_gather(s + NGB - 1, (sr + NGB - 1) % NGB)

        pltpu.make_async_copy(tab_hbm.at[ihs[bank]], gbuf, gsem).wait()

        if sr >= NOB:
            pltpu.make_async_copy(obuf, out_hbm.at[s - NOB, pl.ds(0, JT), wid],
                                  osem).wait()
        else:
            @pl.when(st > 0)
            def _():
                pltpu.make_async_copy(obuf,
                                      out_hbm.at[s - NOB, pl.ds(0, JT), wid],
                                      osem).wait()

        halves = [(idx_v[st, sr, pl.ds(k * L, L)] & 1) * D
                  for k in range(BB // L)]
        pbase = s * D

        def dims(j0, carry):
            jmod = (jnp.full((L,), j0, jnp.int32) + iota) & (D - 1)
            a = pbase + jmod
            pj = plsc.load_gather(pos_v, [a >> 7, a & 127])
            jts, jrs = jmod >> 3, jmod & 7
            for k in range(BB // L):
                x = plsc.load_gather(gbuf, [rowids[k], halves[k] + jmod])
                plsc.store_scatter(obuf, [jts, jrs, rowids[k]], x * SCALE + pj)
            return carry
        lax.fori_loop(0, D, dims, 0, unroll=False)

        pltpu.async_copy(obuf, out_hbm.at[s, pl.ds(0, JT), wid], osem)

    def outer(st, carry):
        for sr in range(8):
            do_block(st, sr)
        return carry

    lax.fori_loop(0, ST, outer, 0, unroll=False)

    for t in range(NOB):
        s = SEQ - NOB + t
        pltpu.make_async_copy(obufs[s % NOB],
                              out_hbm.at[s, pl.ds(0, JT), wid],
                              osems[s % NOB]).wait()


def kernel(inputs, token_table, position_table):
    bsz, seq = inputs.shape
    vocab, d = token_table.shape
    nbb = bsz // BB

    idx_q = inputs.astype(jnp.int32).reshape(nbb, BB, seq // 8, 8).transpose(2, 0, 3, 1)
    tab2 = token_table.reshape(vocab // 2, 2 * d)
    pos2 = position_table.reshape(seq // 2, 2 * d)

    mesh = plsc.VectorSubcoreMesh(core_axis_name="c", subcore_axis_name="s")

    def body(idx_hbm, pos_hbm, tab_hbm, out_hbm, idx_v, pos_v,
             ih0, ih1, ih2, ih3, g0, g1, g2, g3, o0, o1,
             gs0, gs1, gs2, gs3, os0, os1):
        _sc_body(idx_hbm, pos_hbm, tab_hbm, out_hbm, idx_v, pos_v,
                 (ih0, ih1, ih2, ih3), (g0, g1, g2, g3), (o0, o1),
                 (gs0, gs1, gs2, gs3), (os0, os1))

    p5 = pl.kernel(
        body,
        out_type=jax.ShapeDtypeStruct((seq, JT, nbb, 8, BB), jnp.float32),
        mesh=mesh,
        compiler_params=pltpu.CompilerParams(use_tc_tiling_on_sc=True,
                                             needs_layout_passes=False),
        scratch_types=(
            [pltpu.VMEM((ST, 8, BB), jnp.int32),
             pltpu.VMEM((seq // 2, 2 * d), jnp.float32)]
            + [pltpu.VMEM((BB,), jnp.int32) for _ in range(NGB)]
            + [pltpu.VMEM((BB, 2 * d), jnp.float32) for _ in range(NGB)]
            + [pltpu.VMEM((JT, 8, BB), jnp.float32) for _ in range(NOB)]
            + [pltpu.SemaphoreType.DMA for _ in range(NGB + NOB)]
        ),
    )(idx_q, pos2, tab2)

    return p5.transpose(2, 4, 0, 1, 3).reshape(bsz, seq, d)


# speedup vs baseline: 1.6986x; 1.0363x over previous
import functools

import jax
import jax.numpy as jnp
from jax import lax
from jax.experimental import pallas as pl
from jax.experimental.pallas import tpu as pltpu
from jax.experimental.pallas import tpu_sc as plsc

D = 64
SEQ = 200
L = 16
NC = 2
NS = 16
NW = NC * NS

BB = 128
JT = D // 8
ST = SEQ // 8
SCALE = 8.0


NGB = 4
NOB = 2


def _sc_body(idx_hbm, pos_hbm, tab_hbm, out_hbm,
             idx_v, pos_v, ihs, gbufs, obufs, gsems, osems):
    wid = lax.axis_index("s") * NC + lax.axis_index("c")
    pltpu.sync_copy(idx_hbm.at[pl.ds(0, ST), wid], idx_v)
    pltpu.sync_copy(pos_hbm, pos_v)

    iota = lax.iota(jnp.int32, L)
    rowids = [jnp.int32(k * L) + iota for k in range(BB // L)]

    def fire_gather(s, bank):
        st, sr = s // 8, s % 8
        for k in range(BB // L):
            sl = pl.ds(k * L, L)
            ihs[bank][sl] = idx_v[st, sr, sl] >> 1
        pltpu.async_copy(tab_hbm.at[ihs[bank]], gbufs[bank], gsems[bank])

    for b in range(NGB - 1):
        fire_gather(b, b)

    def do_block(st, sr):
        s = st * 8 + sr
        bank = sr % NGB
        gbuf, gsem = gbufs[bank], gsems[bank]
        ob = sr % NOB
        obuf, osem = obufs[ob], osems[ob]

        @pl.when(s + NGB - 1 < SEQ)
        def _():
            fire_gather(s + NGB - 1, (sr + NGB - 1) % NGB)

        pltpu.make_async_copy(tab_hbm.at[ihs[bank]], gbuf, gsem).wait()

        if sr >= NOB:
            pltpu.make_async_copy(obuf, out_hbm.at[s - NOB, pl.ds(0, JT), wid],
                                  osem).wait()
        else:
            @pl.when(st > 0)
            def _():
                pltpu.make_async_copy(obuf,
                                      out_hbm.at[s - NOB, pl.ds(0, JT), wid],
                                      osem).wait()

        halves = [(idx_v[st, sr, pl.ds(k * L, L)] & 1) * D
                  for k in range(BB // L)]
        pbase = s * D

        def dims(j0, carry):
            jmod = (jnp.full((L,), j0, jnp.int32) + iota) & (D - 1)
            a = pbase + jmod
            pj = plsc.load_gather(pos_v, [a >> 7, a & 127])
            jts, jrs = jmod >> 3, jmod & 7
            for k in range(BB // L):
                x = plsc.load_gather(gbuf, [rowids[k], halves[k] + jmod])
                plsc.store_scatter(obuf, [jts, jrs, rowids[k]], x * SCALE + pj)
            return carry
        lax.fori_loop(0, D, dims, 0, unroll=4)

        pltpu.async_copy(obuf, out_hbm.at[s, pl.ds(0, JT), wid], osem)

    def outer(st, carry):
        for sr in range(8):
            do_block(st, sr)
        return carry

    lax.fori_loop(0, ST, outer, 0, unroll=False)

    for t in range(NOB):
        s = SEQ - NOB + t
        pltpu.make_async_copy(obufs[s % NOB],
                              out_hbm.at[s, pl.ds(0, JT), wid],
                              osems[s % NOB]).wait()


def kernel(inputs, token_table, position_table):
    bsz, seq = inputs.shape
    vocab, d = token_table.shape
    nbb = bsz // BB

    idx_q = inputs.astype(jnp.int32).reshape(nbb, BB, seq // 8, 8).transpose(2, 0, 3, 1)
    tab2 = token_table.reshape(vocab // 2, 2 * d)
    pos2 = position_table.reshape(seq // 2, 2 * d)

    mesh = plsc.VectorSubcoreMesh(core_axis_name="c", subcore_axis_name="s")

    def body(idx_hbm, pos_hbm, tab_hbm, out_hbm, idx_v, pos_v,
             ih0, ih1, ih2, ih3, g0, g1, g2, g3, o0, o1,
             gs0, gs1, gs2, gs3, os0, os1):
        _sc_body(idx_hbm, pos_hbm, tab_hbm, out_hbm, idx_v, pos_v,
                 (ih0, ih1, ih2, ih3), (g0, g1, g2, g3), (o0, o1),
                 (gs0, gs1, gs2, gs3), (os0, os1))

    p5 = pl.kernel(
        body,
        out_type=jax.ShapeDtypeStruct((seq, JT, nbb, 8, BB), jnp.float32),
        mesh=mesh,
        compiler_params=pltpu.CompilerParams(use_tc_tiling_on_sc=True,
                                             needs_layout_passes=False),
        scratch_types=(
            [pltpu.VMEM((ST, 8, BB), jnp.int32),
             pltpu.VMEM((seq // 2, 2 * d), jnp.float32)]
            + [pltpu.VMEM((BB,), jnp.int32) for _ in range(NGB)]
            + [pltpu.VMEM((BB, 2 * d), jnp.float32) for _ in range(NGB)]
            + [pltpu.VMEM((JT, 8, BB), jnp.float32) for _ in range(NOB)]
            + [pltpu.SemaphoreType.DMA for _ in range(NGB + NOB)]
        ),
    )(idx_q, pos2, tab2)

    return p5.transpose(2, 4, 0, 1, 3).reshape(bsz, seq, d)


# speedup vs baseline: 1.8236x; 1.0736x over previous
import jax
import jax.numpy as jnp
from jax import lax
from jax.experimental import pallas as pl
from jax.experimental.pallas import tpu as pltpu
from jax.experimental.pallas import tpu_sc as plsc

D = 64
SEQ = 200
L = 16
NC = 2
NS = 16
NW = NC * NS

BB = 128
JT = D // 8
ST = SEQ // 8
SCALE = 8.0

VOCAB = 1000000
NFB = VOCAB // BB
TAIL = VOCAB - NFB * BB
BPW = NFB // NW + 1

NGB = 4
NOB = 2


def _iota_vecs():
    iota = lax.iota(jnp.int32, L)
    rowids = [jnp.int32(k * L) + iota for k in range(BB // L)]
    return iota, rowids


def _relayout_body(tabt_hbm, tail_hbm, scr_hbm,
                   tb0, tb1, tb2, sb0, sb1, si0, si1, si2, so0, so1):
    wid = lax.axis_index("s") * NC + lax.axis_index("c")
    iota, rowids = _iota_vecs()
    rowhalf = [r >> 1 for r in rowids]
    qbase = [(r & 1) * D for r in rowids]
    tbufs, sins = (tb0, tb1, tb2), (si0, si1, si2)
    sbufs, souts = (sb0, sb1), (so0, so1)

    @pl.when(wid == 0)
    def _():
        pltpu.sync_copy(tail_hbm, scr_hbm.at[pl.ds(NFB * D, TAIL // 2)])

    def blk(n):
        return n * NW + wid

    def fire_in(n, bi):
        @pl.when(blk(n) < NFB)
        def _():
            it = blk(n)
            pltpu.async_copy(tabt_hbm.at[pl.ds(0, D), pl.ds(it * BB, BB)],
                             tbufs[bi], sins[bi])

    fire_in(0, 0)
    fire_in(1, 1)

    def do_block(n, bi, bo, has_prev=True):
        it = blk(n)
        fire_in(n + 2, (bi + 2) % 3)

        @pl.when(it < NFB)
        def _():
            pltpu.make_async_copy(
                tabt_hbm.at[pl.ds(0, D), pl.ds(it * BB, BB)],
                tbufs[bi], sins[bi]).wait()

        if has_prev:
            @pl.when(it - 2 * NW < NFB)
            def _():
                pltpu.make_async_copy(
                    sbufs[bo], scr_hbm.at[pl.ds((it - 2 * NW) * (BB // 2), D)],
                    souts[bo]).wait()

        @pl.when(it < NFB)
        def _():
            tbuf, sbuf = tbufs[bi], sbufs[bo]

            def dims(j0, carry):
                jmod = (jnp.full((L,), j0, jnp.int32) + iota) & (D - 1)
                for k in range(BB // L):
                    x = plsc.load_gather(tbuf, [jmod, rowids[k]])
                    plsc.store_scatter(sbuf, [rowhalf[k], qbase[k] + jmod], x)
                return carry
            lax.fori_loop(0, D, dims, 0, unroll=4)

            pltpu.async_copy(sbuf, scr_hbm.at[pl.ds(it * (BB // 2), D)],
                             souts[bo])

    do_block(0, 0, 0, has_prev=False)
    do_block(1, 1, 1, has_prev=False)

    def outer(m, carry):
        for h in range(6):
            n = 2 + m * 6 + h
            do_block(n, (2 + h) % 3, h % 2)
        return carry
    nouter = (BPW - 2) // 6 + 1
    lax.fori_loop(0, nouter, outer, 0, unroll=False)

    nlast = 2 + nouter * 6 - 1
    for n in (nlast - 1, nlast):
        @pl.when(blk(n) < NFB)
        def _():
            pltpu.make_async_copy(
                sbufs[n % 2], scr_hbm.at[pl.ds(blk(n) * (BB // 2), D)],
                souts[n % 2]).wait()


def _gather_body(idx_hbm, pos_hbm, tab_hbm, out_hbm,
                 idx_v, pos_v, g0, g1, g2, g3, o0, o1,
                 gs0, gs1, gs2, gs3, os0, os1):
    wid = lax.axis_index("s") * NC + lax.axis_index("c")
    gbufs, gsems = (g0, g1, g2, g3), (gs0, gs1, gs2, gs3)
    obufs, osems = (o0, o1), (os0, os1)
    iota, rowids = _iota_vecs()

    pltpu.sync_copy(idx_hbm.at[pl.ds(0, ST), wid], idx_v)
    pltpu.sync_copy(pos_hbm, pos_v)

    def fire_gather(s, bank):
        pltpu.async_copy(tab_hbm.at[idx_v.at[s // 8, s % 8]],
                         gbufs[bank], gsems[bank])

    for b in range(NGB - 1):
        fire_gather(b, b)

    def do_block(st, sr):
        s = st * 8 + sr
        bank = sr % NGB
        gbuf, gsem = gbufs[bank], gsems[bank]
        obuf, osem = obufs[sr % NOB], osems[sr % NOB]

        @pl.when(s + NGB - 1 < SEQ)
        def _():
            fire_gather(s + NGB - 1, (sr + NGB - 1) % NGB)

        pltpu.make_async_copy(tab_hbm.at[idx_v.at[st, sr]], gbuf, gsem).wait()

        if sr >= NOB:
            pltpu.make_async_copy(obuf, out_hbm.at[s - NOB, pl.ds(0, JT), wid],
                                  osem).wait()
        else:
            @pl.when(st > 0)
            def _():
                pltpu.make_async_copy(obuf,
                                      out_hbm.at[s - NOB, pl.ds(0, JT), wid],
                                      osem).wait()

        pbase = s * D

        def dims(j0, carry):
            jmod = (jnp.full((L,), j0, jnp.int32) + iota) & (D - 1)
            a = pbase + jmod
            pj = plsc.load_gather(pos_v, [a >> 7, a & 127])
            jts, jrs = jmod >> 3, jmod & 7
            for k in range(BB // L):
                x = plsc.load_gather(gbuf, [rowids[k], jmod])
                plsc.store_scatter(obuf, [jts, jrs, rowids[k]], x * SCALE + pj)
            return carry
        lax.fori_loop(0, D, dims, 0, unroll=4)

        pltpu.async_copy(obuf, out_hbm.at[s, pl.ds(0, JT), wid], osem)

    def outer(st, carry):
        for sr in range(8):
            do_block(st, sr)
        return carry
    lax.fori_loop(0, ST, outer, 0, unroll=False)

    for t in range(NOB):
        s = SEQ - NOB + t
        pltpu.make_async_copy(obufs[s % NOB],
                              out_hbm.at[s, pl.ds(0, JT), wid],
                              osems[s % NOB]).wait()


def kernel(inputs, token_table, position_table):
    bsz, seq = inputs.shape
    vocab, d = token_table.shape
    nbb = bsz // BB

    mesh = plsc.VectorSubcoreMesh(core_axis_name="c", subcore_axis_name="s")

    tabt = token_table.T
    tail2 = token_table[NFB * BB:].reshape(TAIL // 2, 2 * d)

    scr = pl.kernel(
        _relayout_body,
        out_type=jax.ShapeDtypeStruct((vocab // 2, 2 * d), jnp.float32),
        mesh=mesh,
        compiler_params=pltpu.CompilerParams(use_tc_tiling_on_sc=True,
                                             needs_layout_passes=False),
        scratch_types=(
            [pltpu.VMEM((d, BB), jnp.float32) for _ in range(5)]
            + [pltpu.SemaphoreType.DMA for _ in range(5)]
        ),
    )(tabt, tail2)

    tabl = scr.reshape(vocab, d)
    idx_q = inputs.astype(jnp.int32).reshape(nbb, BB, seq // 8, 8).transpose(2, 0, 3, 1)
    pos2 = position_table.reshape(seq // 2, 2 * d)

    p5 = pl.kernel(
        _gather_body,
        out_type=jax.ShapeDtypeStruct((seq, JT, nbb, 8, BB), jnp.float32),
        mesh=mesh,
        compiler_params=pltpu.CompilerParams(use_tc_tiling_on_sc=False,
                                             needs_layout_passes=False),
        scratch_types=(
            [pltpu.VMEM((ST, 8, BB), jnp.int32),
             pltpu.VMEM((seq // 2, 2 * d), jnp.float32)]
            + [pltpu.VMEM((BB, d), jnp.float32) for _ in range(NGB)]
            + [pltpu.VMEM((JT, 8, BB), jnp.float32) for _ in range(NOB)]
            + [pltpu.SemaphoreType.DMA for _ in range(NGB + NOB)]
        ),
    )(idx_q, pos2, tabl)

    return p5.transpose(2, 4, 0, 1, 3).reshape(bsz, seq, d)
